# baseline (speedup 1.0000x reference)
import functools

import jax
import jax.numpy as jnp
from jax import lax
from jax.experimental import pallas as pl
from jax.experimental.pallas import tpu as pltpu
from jax.experimental.pallas import tpu_sc as plsc

_NC = 2
_NS = 16
_NW = _NC * _NS
_CB = 128
_L = 16
_PK = 4


@functools.lru_cache(maxsize=None)
def _make_gather(V, D, Bm, S):
    assert _PK * D == 128 and V % _PK == 0
    btiles = Bm // _CB
    assert btiles == _NW, "one batch tile per subcore"
    mesh = plsc.VectorSubcoreMesh(
        core_axis_name="c", subcore_axis_name="s",
        num_cores=_NC, num_subcores=_NS,
    )

    @functools.partial(
        pl.kernel,
        out_type=jax.ShapeDtypeStruct((S, D, Bm), jnp.float32),
        mesh=mesh,
        scratch_types=[
            pltpu.VMEM((S, _CB), jnp.int32),
            pltpu.VMEM((_CB,), jnp.int32),
            pltpu.VMEM((_CB, _PK * D), jnp.float32),
            pltpu.VMEM((D, _CB), jnp.float32),
            pltpu.SemaphoreType.DMA,
        ],
        compiler_params=pltpu.CompilerParams(
            use_tc_tiling_on_sc=True, needs_layout_passes=False),
    )
    def k(ws_hbm, tbl_hbm, out_hbm, idx_v, q_v, buf, buf_t, sem):
        w = lax.axis_index("s") * _NC + lax.axis_index("c")
        col = w * _CB
        pltpu.sync_copy(ws_hbm.at[:, pl.ds(col, _CB)], idx_v)

        nk = _CB // _L
        rows = [lax.iota(jnp.int32, _L) + kk * _L for kk in range(nk)]

        @pl.loop(0, S)
        def _(s):
            colbase = []
            for kk in range(nk):
                iv = idx_v[s, pl.ds(kk * _L, _L)]
                q_v[pl.ds(kk * _L, _L)] = lax.shift_right_logical(iv, 2)
                colbase.append(lax.shift_left(lax.bitwise_and(iv, 3), 5))
            pltpu.async_copy(tbl_hbm.at[q_v], buf, sem).wait()
            pairs = [(d, kk) for d in range(D) for kk in range(nk)]
            for i in range(0, len(pairs), 8):
                batch = pairs[i:i + 8]
                vs = [
                    plsc.load_gather(buf, [rows[kk], lax.bitwise_or(colbase[kk], d)])
                    for d, kk in batch
                ]
                for (d, kk), v in zip(batch, vs):
                    buf_t[d, pl.ds(kk * _L, _L)] = v
            pltpu.sync_copy(buf_t, out_hbm.at[s, :, pl.ds(col, _CB)])

    return k


def kernel(word_seqs, table):
    Bm, S = word_seqs.shape
    V, D = table.shape
    ws_t = word_seqs.T.astype(jnp.int32)
    tbl_packed = table.reshape(V // _PK, _PK * D)
    out_t = _make_gather(V, D, Bm, S)(ws_t, tbl_packed)
    return out_t.transpose(2, 0, 1)


# speedup vs baseline: 1.5647x; 1.5647x over previous
import functools

import jax
import jax.numpy as jnp
from jax import lax
from jax.experimental import pallas as pl
from jax.experimental.pallas import tpu as pltpu
from jax.experimental.pallas import tpu_sc as plsc

_REPACK_CH = 8192

_NC = 2
_NS = 16
_NW = _NC * _NS
_CB = 128
_L = 16


@functools.lru_cache(maxsize=None)
def _make_repack(V, D):
    grid = (V + _REPACK_CH - 1) // _REPACK_CH

    def body(t_ref, o_ref):
        o_ref[:, 0:D] = t_ref[...].T

    return pl.pallas_call(
        body,
        grid=(grid,),
        in_specs=[pl.BlockSpec((D, _REPACK_CH), lambda g: (0, g))],
        out_specs=pl.BlockSpec((_REPACK_CH, 4 * D), lambda g: (g, 0)),
        out_shape=jax.ShapeDtypeStruct((V, 4 * D), jnp.float32),
    )


@functools.lru_cache(maxsize=None)
def _make_gather(V, D, Bm, S):
    btiles = Bm // _CB
    assert btiles == _NW, "one batch tile per subcore"
    mesh = plsc.VectorSubcoreMesh(
        core_axis_name="c", subcore_axis_name="s",
        num_cores=_NC, num_subcores=_NS,
    )

    @functools.partial(
        pl.kernel,
        out_type=jax.ShapeDtypeStruct((S, D, Bm), jnp.float32),
        mesh=mesh,
        scratch_types=[
            pltpu.VMEM((S, _CB), jnp.int32),
            pltpu.VMEM((_CB, 4 * D), jnp.float32),
            pltpu.VMEM((D, _CB), jnp.float32),
            pltpu.SemaphoreType.DMA,
        ],
        compiler_params=pltpu.CompilerParams(
            use_tc_tiling_on_sc=True, needs_layout_passes=False),
    )
    def k(ws_hbm, tbl_hbm, out_hbm, idx_v, buf, buf_t, sem):
        w = lax.axis_index("s") * _NC + lax.axis_index("c")
        col = w * _CB
        pltpu.sync_copy(ws_hbm.at[:, pl.ds(col, _CB)], idx_v)

        nk = _CB // _L
        rows = [lax.iota(jnp.int32, _L) + kk * _L for kk in range(nk)]

        @pl.loop(0, S)
        def _(s):
            pltpu.async_copy(tbl_hbm.at[idx_v.at[s]], buf, sem).wait()
            pairs = [(d, kk) for d in range(D) for kk in range(nk)]
            for i in range(0, len(pairs), 8):
                batch = pairs[i:i + 8]
                vs = [
                    plsc.load_gather(buf, [rows[kk], jnp.full((_L,), d, jnp.int32)])
                    for d, kk in batch
                ]
                for (d, kk), v in zip(batch, vs):
                    buf_t[d, pl.ds(kk * _L, _L)] = v
            pltpu.sync_copy(buf_t, out_hbm.at[s, :, pl.ds(col, _CB)])

    return k


def kernel(word_seqs, table):
    Bm, S = word_seqs.shape
    V, D = table.shape
    ws_t = word_seqs.T.astype(jnp.int32)
    tbl_padded = _make_repack(V, D)(table.T)
    out_t = _make_gather(V, D, Bm, S)(ws_t, tbl_padded)
    return out_t.transpose(2, 0, 1)


# speedup vs baseline: 1.8146x; 1.1597x over previous
import functools

import jax
import jax.numpy as jnp
from jax import lax
from jax.experimental import pallas as pl
from jax.experimental.pallas import tpu as pltpu
from jax.experimental.pallas import tpu_sc as plsc

_REPACK_CH = 8192

_NC = 2
_NS = 16
_NW = _NC * _NS
_CB = 128
_L = 16


@functools.lru_cache(maxsize=None)
def _make_repack(V, D):
    grid = (V + _REPACK_CH - 1) // _REPACK_CH

    def body(t_ref, o_ref):
        o_ref[:, 0:D] = t_ref[...].T

    return pl.pallas_call(
        body,
        grid=(grid,),
        in_specs=[pl.BlockSpec((D, _REPACK_CH), lambda g: (0, g))],
        out_specs=pl.BlockSpec((_REPACK_CH, 4 * D), lambda g: (g, 0)),
        out_shape=jax.ShapeDtypeStruct((V, 4 * D), jnp.float32),
    )


@functools.lru_cache(maxsize=None)
def _make_gather(V, D, Bm, S):
    btiles = Bm // _CB
    assert btiles == _NW, "one batch tile per subcore"
    mesh = plsc.VectorSubcoreMesh(
        core_axis_name="c", subcore_axis_name="s",
        num_cores=_NC, num_subcores=_NS,
    )

    @functools.partial(
        pl.kernel,
        out_type=jax.ShapeDtypeStruct((S, D, Bm), jnp.float32),
        mesh=mesh,
        scratch_types=[
            pltpu.VMEM((S, _CB), jnp.int32),
            pltpu.VMEM((_CB, 4 * D), jnp.float32),
            pltpu.VMEM((_CB, 4 * D), jnp.float32),
            pltpu.VMEM((D, _CB), jnp.float32),
            pltpu.SemaphoreType.DMA,
            pltpu.SemaphoreType.DMA,
        ],
        compiler_params=pltpu.CompilerParams(
            use_tc_tiling_on_sc=True, needs_layout_passes=False),
    )
    def k(ws_hbm, tbl_hbm, out_hbm, idx_v, buf0, buf1, buf_t, sem0, sem1):
        w = lax.axis_index("s") * _NC + lax.axis_index("c")
        col = w * _CB
        pltpu.sync_copy(ws_hbm.at[:, pl.ds(col, _CB)], idx_v)

        nk = _CB // _L
        rows = [lax.iota(jnp.int32, _L) + kk * _L for kk in range(nk)]
        pairs = [(d, kk) for d in range(D) for kk in range(nk)]

        def drain_extract_write(s, buf, sem):
            pltpu.make_async_copy(tbl_hbm.at[idx_v.at[s]], buf, sem).wait()
            for i in range(0, len(pairs), 8):
                batch = pairs[i:i + 8]
                vs = [
                    plsc.load_gather(buf, [rows[kk], jnp.full((_L,), d, jnp.int32)])
                    for d, kk in batch
                ]
                for (d, kk), v in zip(batch, vs):
                    buf_t[d, pl.ds(kk * _L, _L)] = v
            pltpu.sync_copy(buf_t, out_hbm.at[s, :, pl.ds(col, _CB)])

        def fire(s, buf, sem):
            pltpu.async_copy(tbl_hbm.at[idx_v.at[s]], buf, sem)

        fire(0, buf0, sem0)
        fire(1, buf1, sem1)

        @pl.loop(0, S // 2 - 1)
        def _(g):
            s0 = 2 * g
            drain_extract_write(s0, buf0, sem0)
            fire(s0 + 2, buf0, sem0)
            drain_extract_write(s0 + 1, buf1, sem1)
            fire(s0 + 3, buf1, sem1)

        drain_extract_write(S - 2, buf0, sem0)
        drain_extract_write(S - 1, buf1, sem1)

    return k


def kernel(word_seqs, table):
    Bm, S = word_seqs.shape
    V, D = table.shape
    ws_t = word_seqs.T.astype(jnp.int32)
    tbl_padded = _make_repack(V, D)(table.T)
    out_t = _make_gather(V, D, Bm, S)(ws_t, tbl_padded)
    return out_t.transpose(2, 0, 1)
